# Initial kernel scaffold; baseline (speedup 1.0000x reference)
import jax
import jax.numpy as jnp
from jax.experimental import pallas as pl


def kernel(x, router_w, w1, v1, w2):
    raise NotImplementedError("write your pallas kernel here")


# speedup vs baseline: 1.4335x; 1.4335x over previous
import functools

import jax
import jax.numpy as jnp
from jax.experimental import pallas as pl
from jax.experimental.pallas import tpu as pltpu

B, S, D = 1, 2048, 1024
FFN = 4096
E = 8
TOP_K = 2
T = B * S

BIG = 1e30


def _routing_body(x_ref, rw_ref, gw_ref):
    x = x_ref[...]
    rw = rw_ref[...]
    logits = jax.lax.dot_general(
        x, rw, (((1,), (1,)), ((), ())),
        preferred_element_type=jnp.float32)
    m = jnp.max(logits, axis=1, keepdims=True)
    p = jnp.exp(logits - m)
    g = p / jnp.sum(p, axis=1, keepdims=True)

    idx = jax.lax.broadcasted_iota(jnp.int32, (T, E), 1)
    v1 = jnp.min(g, axis=1, keepdims=True)
    i1 = jnp.min(jnp.where(g == v1, idx, E), axis=1, keepdims=True)
    mask1 = idx == i1
    g2 = jnp.where(mask1, BIG, g)
    v2 = jnp.min(g2, axis=1, keepdims=True)
    i2 = jnp.min(jnp.where(g2 == v2, idx, E), axis=1, keepdims=True)
    mask2 = idx == i2

    denom = jnp.abs(v1) + jnp.abs(v2)
    s1 = v1 / denom
    s2 = v2 / denom
    gw_ref[...] = jnp.where(mask1, s1, 0.0) + jnp.where(mask2, s2, 0.0)


def _routing(x2, router_w):
    return pl.pallas_call(
        _routing_body,
        out_shape=jax.ShapeDtypeStruct((T, E), jnp.float32),
        in_specs=[
            pl.BlockSpec((T, D), lambda: (0, 0)),
            pl.BlockSpec((E, D), lambda: (0, 0)),
        ],
        out_specs=pl.BlockSpec((T, E), lambda: (0, 0)),
    )(x2, router_w)


BM = 1024
BN = 1024
N_N = FFN // BN


def _dense_body(x_ref, gw_ref, w1_ref, v1_ref, w2_ref, out_ref, acc_ref):
    e = pl.program_id(1)
    n = pl.program_id(2)

    @pl.when(jnp.logical_and(e == 0, n == 0))
    def _():
        acc_ref[...] = jnp.zeros_like(acc_ref)

    x = x_ref[...]
    a = jax.lax.dot_general(x, w1_ref[0], (((1,), (1,)), ((), ())),
                            preferred_element_type=jnp.float32)
    b = jax.lax.dot_general(x, v1_ref[0], (((1,), (1,)), ((), ())),
                            preferred_element_type=jnp.float32)
    h = (a * jax.lax.logistic(a)) * b
    y = jax.lax.dot_general(h, w2_ref[0], (((1,), (1,)), ((), ())),
                            preferred_element_type=jnp.float32)

    lane = jax.lax.broadcasted_iota(jnp.int32, (BM, E), 1)
    ge = jnp.sum(jnp.where(lane == e, gw_ref[...], 0.0), axis=1, keepdims=True)
    acc_ref[...] += ge * y

    @pl.when(jnp.logical_and(e == E - 1, n == N_N - 1))
    def _():
        out_ref[...] = acc_ref[...]


def _dense_moe(x2, gw, w1, v1, w2):
    grid = (T // BM, E, N_N)
    return pl.pallas_call(
        _dense_body,
        grid=grid,
        out_shape=jax.ShapeDtypeStruct((T, D), jnp.float32),
        in_specs=[
            pl.BlockSpec((BM, D), lambda t, e, n: (t, 0)),
            pl.BlockSpec((BM, E), lambda t, e, n: (t, 0)),
            pl.BlockSpec((1, BN, D), lambda t, e, n: (e, n, 0)),
            pl.BlockSpec((1, BN, D), lambda t, e, n: (e, n, 0)),
            pl.BlockSpec((1, D, BN), lambda t, e, n: (e, 0, n)),
        ],
        out_specs=pl.BlockSpec((BM, D), lambda t, e, n: (t, 0)),
        scratch_shapes=[pltpu.VMEM((BM, D), jnp.float32)],
        compiler_params=pltpu.CompilerParams(
            dimension_semantics=("parallel", "arbitrary", "arbitrary")),
    )(x2, gw, w1, v1, w2)


def kernel(x, router_w, w1, v1, w2):
    orig_shape = x.shape
    x2 = x.reshape(T, D)
    gw = _routing(x2, router_w)
    y = _dense_moe(x2, gw, w1, v1, w2)
    return y.reshape(orig_shape)
